# Initial kernel scaffold; baseline (speedup 1.0000x reference)
import jax
import jax.numpy as jnp
from jax.experimental import pallas as pl


def kernel(x, edge_index, W1, W2):
    raise NotImplementedError("write your pallas kernel here")


# speedup vs baseline: 3.7765x; 3.7765x over previous
import functools

import jax
import jax.numpy as jnp
from jax import lax
from jax.experimental import pallas as pl
from jax.experimental.pallas import tpu as pltpu
from jax.experimental.pallas import tpu_sc as plsc

N = 10000
E = 160000
D = 256
H = 128
NT = 16
NPAD = 10240
K = 80
EP = E // NT
NCH = EP // K


def _make_sc_agg(compute_deg):
    mesh = plsc.VectorSubcoreMesh(core_axis_name="c", subcore_axis_name="s")
    out_type = [
        jax.ShapeDtypeStruct((NPAD, H), jnp.float32),
        jax.ShapeDtypeStruct((NPAD, H), jnp.float32),
        jax.ShapeDtypeStruct((NPAD,), jnp.float32),
    ]
    scratch = [
        pltpu.VMEM((K,), jnp.int32),
        pltpu.VMEM((K,), jnp.int32),
        pltpu.VMEM((K,), jnp.int32),
        pltpu.VMEM((K, H), jnp.float32),
        pltpu.VMEM((K,), jnp.float32),
        pltpu.VMEM((16, H), jnp.float32),
        pltpu.VMEM((640,), jnp.float32),
        pltpu.VMEM_SHARED((NPAD, H), jnp.float32),
        pltpu.VMEM_SHARED((NPAD,), jnp.float32),
        pltpu.SemaphoreType.DMA,
    ]

    @functools.partial(pl.kernel, out_type=out_type, mesh=mesh,
                       scratch_types=scratch)
    def sc_agg(hcat, src_hbm, dst_hbm, agg0_hbm, agg1_hbm, deg_hbm,
               srcbuf, srcadj, dstbuf, rows, ones, zblk, zvec,
               acc, dacc, sem):
        c = lax.axis_index("c")
        s = lax.axis_index("s")

        z16 = jnp.zeros((16,), jnp.float32)
        for r in range(16):
            for t in range(H // 16):
                zblk[r, pl.ds(t * 16, 16)] = z16
        for t in range(640 // 16):
            zvec[pl.ds(t * 16, 16)] = z16
        one16 = jnp.ones((16,), jnp.float32)
        for t in range(K // 16):
            ones[pl.ds(t * 16, 16)] = one16

        def zbody(i, carry):
            pltpu.sync_copy(zblk, acc.at[pl.ds((s * 40 + i) * 16, 16)])
            return carry
        lax.fori_loop(0, 40, zbody, 0)
        pltpu.sync_copy(zvec, dacc.at[pl.ds(s * 640, 640)])
        plsc.subcore_barrier()

        base = s * EP
        coff = c * N

        def ebody(i, carry):
            off = base + i * K
            pltpu.sync_copy(src_hbm.at[pl.ds(off, K)], srcbuf)
            pltpu.sync_copy(dst_hbm.at[pl.ds(off, K)], dstbuf)
            for t in range(K // 16):
                srcadj[pl.ds(t * 16, 16)] = srcbuf[pl.ds(t * 16, 16)] + coff
            pltpu.async_copy(hcat.at[srcadj], rows, sem).wait()
            pltpu.sync_copy(rows, acc.at[dstbuf], add=True)
            if compute_deg:
                @pl.when(c == 0)
                def _():
                    pltpu.sync_copy(ones, dacc.at[dstbuf], add=True)
            return carry
        lax.fori_loop(0, NCH, ebody, 0)
        plsc.subcore_barrier()

        @pl.when(c == 0)
        def _():
            pltpu.sync_copy(acc.at[pl.ds(s * 640, 640)],
                            agg0_hbm.at[pl.ds(s * 640, 640)])
            pltpu.sync_copy(dacc.at[pl.ds(s * 640, 640)],
                            deg_hbm.at[pl.ds(s * 640, 640)])

        @pl.when(c == 1)
        def _():
            pltpu.sync_copy(acc.at[pl.ds(s * 640, 640)],
                            agg1_hbm.at[pl.ds(s * 640, 640)])

    return sc_agg


_sc_agg_deg = _make_sc_agg(True)
_sc_agg = _make_sc_agg(False)


BN = 1000


def _tc_body(flat, h0, h1, a0, a1, d, ws0, ws1, wn0, wn1, o):
    dv = d[...]
    neigh = dv > 0.0
    r = 1.0 / jnp.maximum(dv, 1.0)
    a0v = jnp.where(neigh, a0[...] * r, h0[...])
    a1v = jnp.where(neigh, a1[...] * r, h1[...])
    acc = (jnp.dot(h0[...], ws0[...], preferred_element_type=jnp.float32)
           + jnp.dot(h1[...], ws1[...], preferred_element_type=jnp.float32)
           + jnp.dot(a0v, wn0[...], preferred_element_type=jnp.float32)
           + jnp.dot(a1v, wn1[...], preferred_element_type=jnp.float32))
    acc = jnp.maximum(acc, 0.0)
    if flat:
        o[...] = acc
    else:
        o[0] = acc[:, :H]
        o[1] = acc[:, H:]


def _make_tc(flat):
    row = lambda i: (i, 0)
    w = pl.BlockSpec((H, D), lambda i: (0, 0))
    in_specs = [
        pl.BlockSpec((BN, H), row),
        pl.BlockSpec((BN, H), row),
        pl.BlockSpec((BN, H), row),
        pl.BlockSpec((BN, H), row),
        pl.BlockSpec((BN, 1), row),
        w, w, w, w,
    ]
    if flat:
        out_spec = pl.BlockSpec((BN, D), row)
        out_shape = jax.ShapeDtypeStruct((N, D), jnp.float32)
    else:
        out_spec = pl.BlockSpec((2, BN, H), lambda i: (0, i, 0))
        out_shape = jax.ShapeDtypeStruct((2, N, H), jnp.float32)
    return pl.pallas_call(
        functools.partial(_tc_body, flat),
        grid=(N // BN,),
        in_specs=in_specs,
        out_specs=out_spec,
        out_shape=out_shape,
    )


_tc_pair = _make_tc(False)
_tc_flat = _make_tc(True)


def kernel(x, edge_index, W1, W2):
    src = edge_index[0].astype(jnp.int32)
    dst = edge_index[1].astype(jnp.int32)
    x0 = x[:, :H]
    x1 = x[:, H:]
    hcat = jnp.concatenate([x0, x1], axis=0)

    agg0, agg1, deg = _sc_agg_deg(hcat, src, dst)
    deg2 = deg.reshape(NPAD, 1)

    W1T = W1.T
    hpair = _tc_pair(x0, x1, agg0, agg1, deg2,
                     W1T[:H], W1T[H:2 * H], W1T[2 * H:3 * H], W1T[3 * H:])

    hcat2 = hpair.reshape(2 * N, H)
    agg0b, agg1b, _ = _sc_agg(hcat2, src, dst)

    W2T = W2.T
    out = _tc_flat(hpair[0], hpair[1], agg0b, agg1b, deg2,
                   W2T[:H], W2T[H:2 * H], W2T[2 * H:3 * H], W2T[3 * H:])
    return out


# speedup vs baseline: 8.2722x; 2.1905x over previous
import functools

import jax
import jax.numpy as jnp
from jax import lax
from jax.experimental import pallas as pl
from jax.experimental.pallas import tpu as pltpu
from jax.experimental.pallas import tpu_sc as plsc

N = 10000
E = 160000
D = 256
H = 128
NT = 16
NPAD = 10240
K = 80
EP = E // NT
NCH = EP // K


def _make_sc_agg(compute_deg):
    mesh = plsc.VectorSubcoreMesh(core_axis_name="c", subcore_axis_name="s")
    out_type = [
        jax.ShapeDtypeStruct((NPAD, H), jnp.float32),
        jax.ShapeDtypeStruct((NPAD, H), jnp.float32),
        jax.ShapeDtypeStruct((NPAD,), jnp.float32),
    ]
    scratch = [
        pltpu.VMEM((EP,), jnp.int32),
        pltpu.VMEM((EP,), jnp.int32),
        pltpu.VMEM((K,), jnp.int32),
        pltpu.VMEM((K, H), jnp.float32),
        pltpu.VMEM((K, H), jnp.float32),
        pltpu.VMEM((K,), jnp.float32),
        pltpu.VMEM((16, H), jnp.float32),
        pltpu.VMEM((640,), jnp.float32),
        pltpu.VMEM_SHARED((NPAD, H), jnp.float32),
        pltpu.VMEM_SHARED((NPAD,), jnp.float32),
        pltpu.SemaphoreType.DMA,
        pltpu.SemaphoreType.DMA,
    ]

    @functools.partial(pl.kernel, out_type=out_type, mesh=mesh,
                       scratch_types=scratch)
    def sc_agg(hcat, srcc_hbm, dst_hbm, agg0_hbm, agg1_hbm, deg_hbm,
               srcall, dstall, dstbuf, rows0, rows1, ones, zblk, zvec,
               acc, dacc, sem0, sem1):
        c = lax.axis_index("c")
        s = lax.axis_index("s")

        pltpu.sync_copy(srcc_hbm.at[pl.ds(c * E + s * EP, EP)], srcall)
        pltpu.sync_copy(dst_hbm.at[pl.ds(s * EP, EP)], dstall)

        z16 = jnp.zeros((16,), jnp.float32)
        for r in range(16):
            for t in range(H // 16):
                zblk[r, pl.ds(t * 16, 16)] = z16
        for t in range(640 // 16):
            zvec[pl.ds(t * 16, 16)] = z16
        one16 = jnp.ones((16,), jnp.float32)
        for t in range(K // 16):
            ones[pl.ds(t * 16, 16)] = one16

        def zbody(i, carry):
            pltpu.sync_copy(zblk, acc.at[pl.ds((s * 40 + i) * 16, 16)])
            return carry
        lax.fori_loop(0, 40, zbody, 0)
        pltpu.sync_copy(zvec, dacc.at[pl.ds(s * 640, 640)])
        plsc.subcore_barrier()

        def gather(i, rows, sem):
            return pltpu.async_copy(
                hcat.at[srcall.at[pl.ds(i * K, K)]], rows, sem)

        gather(0, rows0, sem0)

        def scat(i, rows):
            for t in range(K // 16):
                dstbuf[pl.ds(t * 16, 16)] = dstall[pl.ds(i * K + t * 16, 16)]
            pltpu.sync_copy(rows, acc.at[dstbuf], add=True)
            if compute_deg:
                @pl.when(c == 0)
                def _():
                    pltpu.sync_copy(ones, dacc.at[dstbuf], add=True)

        def ebody(i, carry):
            nxt = i + 1

            @pl.when(jnp.logical_and(nxt < NCH, lax.rem(nxt, 2) == 1))
            def _():
                gather(nxt, rows1, sem1)

            @pl.when(jnp.logical_and(nxt < NCH, lax.rem(nxt, 2) == 0))
            def _():
                gather(nxt, rows0, sem0)

            @pl.when(lax.rem(i, 2) == 0)
            def _():
                pltpu.make_async_copy(
                    hcat.at[srcall.at[pl.ds(i * K, K)]], rows0, sem0).wait()
                scat(i, rows0)

            @pl.when(lax.rem(i, 2) == 1)
            def _():
                pltpu.make_async_copy(
                    hcat.at[srcall.at[pl.ds(i * K, K)]], rows1, sem1).wait()
                scat(i, rows1)
            return carry
        lax.fori_loop(0, NCH, ebody, 0)
        plsc.subcore_barrier()

        @pl.when(c == 0)
        def _():
            pltpu.sync_copy(acc.at[pl.ds(s * 640, 640)],
                            agg0_hbm.at[pl.ds(s * 640, 640)])
            pltpu.sync_copy(dacc.at[pl.ds(s * 640, 640)],
                            deg_hbm.at[pl.ds(s * 640, 640)])

        @pl.when(c == 1)
        def _():
            pltpu.sync_copy(acc.at[pl.ds(s * 640, 640)],
                            agg1_hbm.at[pl.ds(s * 640, 640)])

    return sc_agg


_sc_agg_deg = _make_sc_agg(True)
_sc_agg = _make_sc_agg(False)


BN = 1000


def _tc_body(flat, h0, h1, a0, a1, d, ws0, ws1, wn0, wn1, o):
    dv = d[...]
    neigh = dv > 0.0
    r = 1.0 / jnp.maximum(dv, 1.0)
    a0v = jnp.where(neigh, a0[...] * r, h0[...])
    a1v = jnp.where(neigh, a1[...] * r, h1[...])
    acc = (jnp.dot(h0[...], ws0[...], preferred_element_type=jnp.float32)
           + jnp.dot(h1[...], ws1[...], preferred_element_type=jnp.float32)
           + jnp.dot(a0v, wn0[...], preferred_element_type=jnp.float32)
           + jnp.dot(a1v, wn1[...], preferred_element_type=jnp.float32))
    acc = jnp.maximum(acc, 0.0)
    if flat:
        o[...] = acc
    else:
        o[0] = acc[:, :H]
        o[1] = acc[:, H:]


def _make_tc(flat):
    row = lambda i: (i, 0)
    w = pl.BlockSpec((H, D), lambda i: (0, 0))
    in_specs = [
        pl.BlockSpec((BN, H), row),
        pl.BlockSpec((BN, H), row),
        pl.BlockSpec((BN, H), row),
        pl.BlockSpec((BN, H), row),
        pl.BlockSpec((BN, 1), row),
        w, w, w, w,
    ]
    if flat:
        out_spec = pl.BlockSpec((BN, D), row)
        out_shape = jax.ShapeDtypeStruct((N, D), jnp.float32)
    else:
        out_spec = pl.BlockSpec((2, BN, H), lambda i: (0, i, 0))
        out_shape = jax.ShapeDtypeStruct((2, N, H), jnp.float32)
    return pl.pallas_call(
        functools.partial(_tc_body, flat),
        grid=(N // BN,),
        in_specs=in_specs,
        out_specs=out_spec,
        out_shape=out_shape,
    )


_tc_pair = _make_tc(False)
_tc_flat = _make_tc(True)


def kernel(x, edge_index, W1, W2):
    src = edge_index[0].astype(jnp.int32)
    dst = edge_index[1].astype(jnp.int32)
    srcc = jnp.concatenate([src, src + jnp.int32(N)])
    x0 = x[:, :H]
    x1 = x[:, H:]
    hcat = jnp.concatenate([x0, x1], axis=0)

    agg0, agg1, deg = _sc_agg_deg(hcat, srcc, dst)
    deg2 = deg.reshape(NPAD, 1)

    W1T = W1.T
    hpair = _tc_pair(x0, x1, agg0, agg1, deg2,
                     W1T[:H], W1T[H:2 * H], W1T[2 * H:3 * H], W1T[3 * H:])

    hcat2 = hpair.reshape(2 * N, H)
    agg0b, agg1b, _ = _sc_agg(hcat2, srcc, dst)

    W2T = W2.T
    out = _tc_flat(hpair[0], hpair[1], agg0b, agg1b, deg2,
                   W2T[:H], W2T[H:2 * H], W2T[2 * H:3 * H], W2T[3 * H:])
    return out
